# baseline (speedup 1.0000x reference)
import jax
import jax.numpy as jnp
from jax import lax
from jax.experimental import pallas as pl
from jax.experimental.pallas import tpu as pltpu
from jax.experimental.pallas import tpu_sc as plsc

N = 10000
D = 128
DH = 64
E = 320000

NC = 2
NS = 16
CH = 128
NH = 4
HGPT = 40
EPT = CH * NH * HGPT
E_PAD = NS * EPT
NZ = 10240
RPT = NZ // NS
NRS = 624
DUMMY = N

BR = 1000
BN_S = (1.0 + 1e-5) ** -0.5

_SC_PARAMS = pltpu.CompilerParams(use_tc_tiling_on_sc=False)


def _deg_body(dstr_hbm, zeros_hbm, ones_hbm, hist_out, hist_sh, idx_d,
              ones_v, sem):
    c = lax.axis_index("c")
    s = lax.axis_index("s")
    r0 = s * RPT
    pltpu.sync_copy(zeros_hbm, hist_sh.at[pl.ds(r0, RPT)])
    pltpu.sync_copy(ones_hbm, ones_v)
    plsc.subcore_barrier()

    for j in range(NH // 2):
        pltpu.sync_copy(dstr_hbm.at[s, (NH // 2) * c + j], idx_d)

        def step(g, carry):
            pltpu.async_copy(ones_v, hist_sh.at[idx_d.at[g]], sem,
                             add=True)
            return carry

        lax.fori_loop(0, HGPT, step, 0)

        def drain(g, carry):
            pltpu.make_async_copy(ones_v, hist_sh.at[idx_d.at[0]],
                                  sem).wait()
            return carry

        lax.fori_loop(0, HGPT, drain, 0)

    plsc.subcore_barrier()
    pltpu.sync_copy(hist_sh.at[pl.ds(r0, RPT)],
                    hist_out.at[c, pl.ds(r0, RPT)])


def _sc_deg(dstr, zeros64, ones64):
    mesh = plsc.VectorSubcoreMesh(core_axis_name="c", subcore_axis_name="s")
    fn = pl.kernel(
        _deg_body,
        mesh=mesh,
        compiler_params=_SC_PARAMS,
        out_type=jax.ShapeDtypeStruct((NC, NZ, DH), jnp.float32),
        scratch_types=[
            pltpu.VMEM_SHARED((NZ, DH), jnp.float32),
            pltpu.VMEM((HGPT, CH), jnp.int32),
            pltpu.VMEM((CH, DH), jnp.float32),
            pltpu.SemaphoreType.DMA,
        ],
    )
    return fn(dstr, zeros64, ones64)


def _agg_body(y2_hbm, srcr_hbm, dstr_hbm, zeros_hbm, z_out, y_sh, z_sh,
              idx_s, idx_d, rows0, rows1, rows2, rows3, sem0, sem1, sem2,
              sem3, sem4, sem5, sem6, sem7):
    c = lax.axis_index("c")
    s = lax.axis_index("s")
    r0 = s * RPT
    pltpu.sync_copy(zeros_hbm, z_sh.at[pl.ds(r0, RPT)])
    pltpu.sync_copy(y2_hbm.at[c, pl.ds(s * NRS, NRS)],
                    y_sh.at[pl.ds(s * NRS, NRS)])

    @pl.when(s == NS - 1)
    def _():
        pltpu.sync_copy(y2_hbm.at[c, pl.ds(NS * NRS, N - NS * NRS)],
                        y_sh.at[pl.ds(NS * NRS, N - NS * NRS)])

    plsc.subcore_barrier()

    rows = [rows0, rows1, rows2, rows3]
    gsems = [sem0, sem1, sem2, sem3]
    ssems = [sem4, sem5, sem6, sem7]

    for h in range(NH):
        pltpu.sync_copy(srcr_hbm.at[s, h], idx_s)
        pltpu.sync_copy(dstr_hbm.at[s, h], idx_d)
        for k in range(3):
            pltpu.async_copy(y_sh.at[idx_s.at[k]], rows[k], gsems[k])

        def quad(i, carry):
            for k in range(4):
                g = 4 * i + k
                kn = (k + 3) % 4
                pltpu.make_async_copy(y_sh.at[idx_s.at[g]], rows[k],
                                      gsems[k]).wait()
                pltpu.async_copy(rows[k], z_sh.at[idx_d.at[g]], ssems[k],
                                 add=True)

                @pl.when((g >= 1) & (g + 3 < HGPT))
                def _():
                    pltpu.make_async_copy(rows[kn], z_sh.at[idx_d.at[g]],
                                          ssems[kn]).wait()

                @pl.when(g + 3 < HGPT)
                def _():
                    pltpu.async_copy(y_sh.at[idx_s.at[g + 3]], rows[kn],
                                     gsems[kn])
            return carry

        lax.fori_loop(0, HGPT // 4, quad, 0)
        for k in range(4):
            pltpu.make_async_copy(rows[k], z_sh.at[idx_d.at[0]],
                                  ssems[k]).wait()

    plsc.subcore_barrier()
    pltpu.sync_copy(z_sh.at[pl.ds(r0, RPT)], z_out.at[c, pl.ds(r0, RPT)])


def _sc_agg(y2, srcr, dstr, zeros64):
    mesh = plsc.VectorSubcoreMesh(core_axis_name="c", subcore_axis_name="s")
    fn = pl.kernel(
        _agg_body,
        mesh=mesh,
        compiler_params=_SC_PARAMS,
        out_type=jax.ShapeDtypeStruct((NC, NZ, DH), jnp.float32),
        scratch_types=[
            pltpu.VMEM_SHARED((N, DH), jnp.float32),
            pltpu.VMEM_SHARED((NZ, DH), jnp.float32),
            pltpu.VMEM((HGPT, CH), jnp.int32),
            pltpu.VMEM((HGPT, CH), jnp.int32),
            pltpu.VMEM((CH, DH), jnp.float32),
            pltpu.VMEM((CH, DH), jnp.float32),
            pltpu.VMEM((CH, DH), jnp.float32),
            pltpu.VMEM((CH, DH), jnp.float32),
            pltpu.SemaphoreType.DMA,
            pltpu.SemaphoreType.DMA,
            pltpu.SemaphoreType.DMA,
            pltpu.SemaphoreType.DMA,
            pltpu.SemaphoreType.DMA,
            pltpu.SemaphoreType.DMA,
            pltpu.SemaphoreType.DMA,
            pltpu.SemaphoreType.DMA,
        ],
    )
    return fn(y2, srcr, dstr, zeros64)


def _prep_body(hist_ref, x_ref, w_ref, dinv_ref, y2_ref):
    deg = hist_ref[0, :, 0:1] + hist_ref[1, :, 0:1] + 1.0
    dv = lax.rsqrt(deg)
    dinv_ref[...] = jnp.broadcast_to(dv, (BR, 16))
    y = jnp.dot(x_ref[...], w_ref[...],
                preferred_element_type=jnp.float32) * dv
    y2_ref[0] = y[:, :DH]
    y2_ref[1] = y[:, DH:]


def _tc_prep(hist, x, w):
    return pl.pallas_call(
        _prep_body,
        grid=(N // BR,),
        in_specs=[
            pl.BlockSpec((NC, BR, DH), lambda i: (0, i, 0)),
            pl.BlockSpec((BR, D), lambda i: (i, 0)),
            pl.BlockSpec((D, D), lambda i: (0, 0)),
        ],
        out_specs=[
            pl.BlockSpec((BR, 16), lambda i: (i, 0)),
            pl.BlockSpec((NC, BR, DH), lambda i: (0, i, 0)),
        ],
        out_shape=[
            jax.ShapeDtypeStruct((N, 16), jnp.float32),
            jax.ShapeDtypeStruct((NC, N, DH), jnp.float32),
        ],
    )(hist, x, w)


def _stage_body(z_ref, y_ref, dinv_ref, w_ref, b_ref, g_ref, be_ref,
                yo_ref):
    dv = dinv_ref[:, 0:1]
    zc = jnp.concatenate([z_ref[0], z_ref[1]], axis=1)
    yc = jnp.concatenate([y_ref[0], y_ref[1]], axis=1)
    t = (zc + yc) * dv + b_ref[...]
    h = jnp.maximum(t * (BN_S * g_ref[...]) + be_ref[...], 0.0)
    yn = jnp.dot(h, w_ref[...], preferred_element_type=jnp.float32) * dv
    yo_ref[0] = yn[:, :DH]
    yo_ref[1] = yn[:, DH:]


def _tc_stage(z, y2, dinv16, w_next, b, g, be):
    return pl.pallas_call(
        _stage_body,
        grid=(N // BR,),
        in_specs=[
            pl.BlockSpec((NC, BR, DH), lambda i: (0, i, 0)),
            pl.BlockSpec((NC, BR, DH), lambda i: (0, i, 0)),
            pl.BlockSpec((BR, 16), lambda i: (i, 0)),
            pl.BlockSpec((D, D), lambda i: (0, 0)),
            pl.BlockSpec((1, D), lambda i: (0, 0)),
            pl.BlockSpec((1, D), lambda i: (0, 0)),
            pl.BlockSpec((1, D), lambda i: (0, 0)),
        ],
        out_specs=pl.BlockSpec((NC, BR, DH), lambda i: (0, i, 0)),
        out_shape=jax.ShapeDtypeStruct((NC, N, DH), jnp.float32),
    )(z, y2, dinv16, w_next, b.reshape(1, D), g.reshape(1, D),
      be.reshape(1, D))


def _final_body(z_ref, y_ref, dinv_ref, b_ref, o_ref):
    dv = dinv_ref[:, 0:1]
    zc = jnp.concatenate([z_ref[0], z_ref[1]], axis=1)
    yc = jnp.concatenate([y_ref[0], y_ref[1]], axis=1)
    o = (zc + yc) * dv + b_ref[...]
    m = jnp.max(o, axis=1, keepdims=True)
    lse = jnp.log(jnp.sum(jnp.exp(o - m), axis=1, keepdims=True)) + m
    o_ref[...] = o - lse


def _tc_final(z, y2, dinv16, b):
    return pl.pallas_call(
        _final_body,
        grid=(N // BR,),
        in_specs=[
            pl.BlockSpec((NC, BR, DH), lambda i: (0, i, 0)),
            pl.BlockSpec((NC, BR, DH), lambda i: (0, i, 0)),
            pl.BlockSpec((BR, 16), lambda i: (i, 0)),
            pl.BlockSpec((1, D), lambda i: (0, 0)),
        ],
        out_specs=pl.BlockSpec((BR, D), lambda i: (i, 0)),
        out_shape=jax.ShapeDtypeStruct((N, D), jnp.float32),
    )(z, y2, dinv16, b.reshape(1, D))


def kernel(x, edge_index, relations, W1, b1, g1, be1, W2, b2, g2, be2,
           W3, b3):
    del relations
    pad = E_PAD - E
    src = jnp.concatenate(
        [edge_index[0], jnp.zeros((pad,), jnp.int32)]).reshape(
            NS, NH, HGPT, CH)
    dst = jnp.concatenate(
        [edge_index[1], jnp.full((pad,), DUMMY, jnp.int32)]).reshape(
            NS, NH, HGPT, CH)
    zeros64 = jnp.zeros((RPT, DH), jnp.float32)
    ones64 = jnp.ones((CH, DH), jnp.float32)

    hist = _sc_deg(dst, zeros64, ones64)
    dinv16, y1 = _tc_prep(hist, x, W1)
    z1 = _sc_agg(y1, src, dst, zeros64)
    y2 = _tc_stage(z1, y1, dinv16, W2, b1, g1, be1)
    z2 = _sc_agg(y2, src, dst, zeros64)
    y3 = _tc_stage(z2, y2, dinv16, W3, b2, g2, be2)
    z3 = _sc_agg(y3, src, dst, zeros64)
    return _tc_final(z3, y3, dinv16, b3)


# speedup vs baseline: 1.1741x; 1.1741x over previous
import jax
import jax.numpy as jnp
from jax import lax
from jax.experimental import pallas as pl
from jax.experimental.pallas import tpu as pltpu
from jax.experimental.pallas import tpu_sc as plsc

N = 10000
D = 128
DH = 64
E = 320000

NC = 2
NS = 16
CH = 128
NH = 4
HGPT = 40
EPT = CH * NH * HGPT
E_PAD = NS * EPT
NZ = 10240
RPT = NZ // NS
NRS = 624
DUMMY = N

BR = 2000
BN_S = (1.0 + 1e-5) ** -0.5

_SC_PARAMS = pltpu.CompilerParams(use_tc_tiling_on_sc=False)


def _deg_body(dstr_hbm, zeros_hbm, ones_hbm, hist_out, hist_sh, idx_d,
              ones_v, sem):
    c = lax.axis_index("c")
    s = lax.axis_index("s")
    r0 = s * RPT
    pltpu.sync_copy(zeros_hbm, hist_sh.at[pl.ds(r0, RPT)])
    pltpu.sync_copy(ones_hbm, ones_v)
    plsc.subcore_barrier()

    for j in range(NH // 2):
        pltpu.sync_copy(dstr_hbm.at[s, (NH // 2) * c + j], idx_d)

        def step(g, carry):
            pltpu.async_copy(ones_v, hist_sh.at[idx_d.at[g]], sem,
                             add=True)
            return carry

        lax.fori_loop(0, HGPT, step, 0)

        def drain(g, carry):
            pltpu.make_async_copy(ones_v, hist_sh.at[idx_d.at[0]],
                                  sem).wait()
            return carry

        lax.fori_loop(0, HGPT, drain, 0)

    plsc.subcore_barrier()
    pltpu.sync_copy(hist_sh.at[pl.ds(r0, RPT)],
                    hist_out.at[pl.ds(r0, RPT), pl.ds(c * DH, DH)])


def _sc_deg(dstr, zeros64, ones64):
    mesh = plsc.VectorSubcoreMesh(core_axis_name="c", subcore_axis_name="s")
    fn = pl.kernel(
        _deg_body,
        mesh=mesh,
        compiler_params=_SC_PARAMS,
        out_type=jax.ShapeDtypeStruct((NZ, D), jnp.float32),
        scratch_types=[
            pltpu.VMEM_SHARED((NZ, DH), jnp.float32),
            pltpu.VMEM((HGPT, CH), jnp.int32),
            pltpu.VMEM((CH, DH), jnp.float32),
            pltpu.SemaphoreType.DMA,
        ],
    )
    return fn(dstr, zeros64, ones64)


def _agg_body(y_hbm, srcr_hbm, dstr_hbm, zeros_hbm, z_out, y_sh, z_sh,
              idx_s, idx_d, rows0, rows1, rows2, rows3, sem0, sem1, sem2,
              sem3, sem4, sem5, sem6, sem7):
    c = lax.axis_index("c")
    s = lax.axis_index("s")
    r0 = s * RPT
    c0 = c * DH
    pltpu.sync_copy(zeros_hbm, z_sh.at[pl.ds(r0, RPT)])
    pltpu.sync_copy(y_hbm.at[pl.ds(s * NRS, NRS), pl.ds(c0, DH)],
                    y_sh.at[pl.ds(s * NRS, NRS)])

    @pl.when(s == NS - 1)
    def _():
        pltpu.sync_copy(
            y_hbm.at[pl.ds(NS * NRS, N - NS * NRS), pl.ds(c0, DH)],
            y_sh.at[pl.ds(NS * NRS, N - NS * NRS)])

    plsc.subcore_barrier()

    rows = [rows0, rows1, rows2, rows3]
    gsems = [sem0, sem1, sem2, sem3]
    ssems = [sem4, sem5, sem6, sem7]

    for h in range(NH):
        pltpu.sync_copy(srcr_hbm.at[s, h], idx_s)
        pltpu.sync_copy(dstr_hbm.at[s, h], idx_d)
        for k in range(3):
            pltpu.async_copy(y_sh.at[idx_s.at[k]], rows[k], gsems[k])

        def quad(i, carry):
            for k in range(4):
                g = 4 * i + k
                kn = (k + 3) % 4
                pltpu.make_async_copy(y_sh.at[idx_s.at[g]], rows[k],
                                      gsems[k]).wait()
                pltpu.async_copy(rows[k], z_sh.at[idx_d.at[g]], ssems[k],
                                 add=True)

                @pl.when((g >= 1) & (g + 3 < HGPT))
                def _():
                    pltpu.make_async_copy(rows[kn], z_sh.at[idx_d.at[g]],
                                          ssems[kn]).wait()

                @pl.when(g + 3 < HGPT)
                def _():
                    pltpu.async_copy(y_sh.at[idx_s.at[g + 3]], rows[kn],
                                     gsems[kn])
            return carry

        lax.fori_loop(0, HGPT // 4, quad, 0)
        for k in range(4):
            pltpu.make_async_copy(rows[k], z_sh.at[idx_d.at[0]],
                                  ssems[k]).wait()

    plsc.subcore_barrier()
    pltpu.sync_copy(z_sh.at[pl.ds(r0, RPT)],
                    z_out.at[pl.ds(r0, RPT), pl.ds(c0, DH)])


def _sc_agg(y, srcr, dstr, zeros64):
    mesh = plsc.VectorSubcoreMesh(core_axis_name="c", subcore_axis_name="s")
    fn = pl.kernel(
        _agg_body,
        mesh=mesh,
        compiler_params=_SC_PARAMS,
        out_type=jax.ShapeDtypeStruct((NZ, D), jnp.float32),
        scratch_types=[
            pltpu.VMEM_SHARED((N, DH), jnp.float32),
            pltpu.VMEM_SHARED((NZ, DH), jnp.float32),
            pltpu.VMEM((HGPT, CH), jnp.int32),
            pltpu.VMEM((HGPT, CH), jnp.int32),
            pltpu.VMEM((CH, DH), jnp.float32),
            pltpu.VMEM((CH, DH), jnp.float32),
            pltpu.VMEM((CH, DH), jnp.float32),
            pltpu.VMEM((CH, DH), jnp.float32),
            pltpu.SemaphoreType.DMA,
            pltpu.SemaphoreType.DMA,
            pltpu.SemaphoreType.DMA,
            pltpu.SemaphoreType.DMA,
            pltpu.SemaphoreType.DMA,
            pltpu.SemaphoreType.DMA,
            pltpu.SemaphoreType.DMA,
            pltpu.SemaphoreType.DMA,
        ],
    )
    return fn(y, srcr, dstr, zeros64)


def _prep_body(hist_ref, x_ref, w_ref, dinv_ref, y_ref):
    deg = hist_ref[:, 0:1] + hist_ref[:, DH:DH + 1] + 1.0
    dv = lax.rsqrt(deg)
    dinv_ref[...] = jnp.broadcast_to(dv, (BR, 16))
    y_ref[...] = jnp.dot(x_ref[...], w_ref[...],
                         preferred_element_type=jnp.float32) * dv


def _tc_prep(hist, x, w):
    return pl.pallas_call(
        _prep_body,
        grid=(N // BR,),
        in_specs=[
            pl.BlockSpec((BR, D), lambda i: (i, 0)),
            pl.BlockSpec((BR, D), lambda i: (i, 0)),
            pl.BlockSpec((D, D), lambda i: (0, 0)),
        ],
        out_specs=[
            pl.BlockSpec((BR, 16), lambda i: (i, 0)),
            pl.BlockSpec((BR, D), lambda i: (i, 0)),
        ],
        out_shape=[
            jax.ShapeDtypeStruct((N, 16), jnp.float32),
            jax.ShapeDtypeStruct((N, D), jnp.float32),
        ],
    )(hist, x, w)


def _stage_body(z_ref, y_ref, dinv_ref, w_ref, b_ref, g_ref, be_ref,
                yo_ref):
    dv = dinv_ref[:, 0:1]
    t = (z_ref[...] + y_ref[...]) * dv + b_ref[...]
    h = jnp.maximum(t * (BN_S * g_ref[...]) + be_ref[...], 0.0)
    yo_ref[...] = jnp.dot(h, w_ref[...],
                          preferred_element_type=jnp.float32) * dv


def _tc_stage(z, y, dinv16, w_next, b, g, be):
    return pl.pallas_call(
        _stage_body,
        grid=(N // BR,),
        in_specs=[
            pl.BlockSpec((BR, D), lambda i: (i, 0)),
            pl.BlockSpec((BR, D), lambda i: (i, 0)),
            pl.BlockSpec((BR, 16), lambda i: (i, 0)),
            pl.BlockSpec((D, D), lambda i: (0, 0)),
            pl.BlockSpec((1, D), lambda i: (0, 0)),
            pl.BlockSpec((1, D), lambda i: (0, 0)),
            pl.BlockSpec((1, D), lambda i: (0, 0)),
        ],
        out_specs=pl.BlockSpec((BR, D), lambda i: (i, 0)),
        out_shape=jax.ShapeDtypeStruct((N, D), jnp.float32),
    )(z, y, dinv16, w_next, b.reshape(1, D), g.reshape(1, D),
      be.reshape(1, D))


def _final_body(z_ref, y_ref, dinv_ref, b_ref, o_ref):
    dv = dinv_ref[:, 0:1]
    o = (z_ref[...] + y_ref[...]) * dv + b_ref[...]
    m = jnp.max(o, axis=1, keepdims=True)
    lse = jnp.log(jnp.sum(jnp.exp(o - m), axis=1, keepdims=True)) + m
    o_ref[...] = o - lse


def _tc_final(z, y, dinv16, b):
    return pl.pallas_call(
        _final_body,
        grid=(N // BR,),
        in_specs=[
            pl.BlockSpec((BR, D), lambda i: (i, 0)),
            pl.BlockSpec((BR, D), lambda i: (i, 0)),
            pl.BlockSpec((BR, 16), lambda i: (i, 0)),
            pl.BlockSpec((1, D), lambda i: (0, 0)),
        ],
        out_specs=pl.BlockSpec((BR, D), lambda i: (i, 0)),
        out_shape=jax.ShapeDtypeStruct((N, D), jnp.float32),
    )(z, y, dinv16, b.reshape(1, D))


def kernel(x, edge_index, relations, W1, b1, g1, be1, W2, b2, g2, be2,
           W3, b3):
    del relations
    pad = E_PAD - E
    src = jnp.concatenate(
        [edge_index[0], jnp.zeros((pad,), jnp.int32)]).reshape(
            NS, NH, HGPT, CH)
    dst = jnp.concatenate(
        [edge_index[1], jnp.full((pad,), DUMMY, jnp.int32)]).reshape(
            NS, NH, HGPT, CH)
    zeros64 = jnp.zeros((RPT, DH), jnp.float32)
    ones64 = jnp.ones((CH, DH), jnp.float32)

    hist = _sc_deg(dst, zeros64, ones64)
    dinv16, y1 = _tc_prep(hist, x, W1)
    z1 = _sc_agg(y1, src, dst, zeros64)
    y2 = _tc_stage(z1, y1, dinv16, W2, b1, g1, be1)
    z2 = _sc_agg(y2, src, dst, zeros64)
    y3 = _tc_stage(z2, y2, dinv16, W3, b2, g2, be2)
    z3 = _sc_agg(y3, src, dst, zeros64)
    return _tc_final(z3, y3, dinv16, b3)


# speedup vs baseline: 1.2882x; 1.0972x over previous
import jax
import jax.numpy as jnp
from jax import lax
from jax.experimental import pallas as pl
from jax.experimental.pallas import tpu as pltpu
from jax.experimental.pallas import tpu_sc as plsc

N = 10000
D = 128
DH = 64
E = 320000

NC = 2
NS = 16
CH = 125
NH = 4
HGPT = 40
EPT = CH * NH * HGPT
NZ = 10240
RPT = NZ // NS
NRS = 624
DHIST = 16

BR = 2000
BN_S = (1.0 + 1e-5) ** -0.5

_SC_PARAMS = pltpu.CompilerParams(use_tc_tiling_on_sc=False)


def _deg_body(dstr_hbm, zeros_hbm, ones_hbm, hist_out, hist_sh, idx_d,
              ones_v, sem):
    c = lax.axis_index("c")
    s = lax.axis_index("s")
    r0 = s * RPT
    pltpu.sync_copy(zeros_hbm, hist_sh.at[pl.ds(r0, RPT)])
    pltpu.sync_copy(ones_hbm, ones_v)
    plsc.subcore_barrier()

    for j in range(NH // 2):
        pltpu.sync_copy(dstr_hbm.at[s, (NH // 2) * c + j], idx_d)

        def step(g, carry):
            pltpu.async_copy(ones_v, hist_sh.at[idx_d.at[g]], sem,
                             add=True)
            return carry

        lax.fori_loop(0, HGPT, step, 0)

        def drain(g, carry):
            pltpu.make_async_copy(ones_v, hist_sh.at[idx_d.at[0]],
                                  sem).wait()
            return carry

        lax.fori_loop(0, HGPT, drain, 0)

    plsc.subcore_barrier()
    pltpu.sync_copy(hist_sh.at[pl.ds(r0, RPT)],
                    hist_out.at[pl.ds(r0, RPT), pl.ds(c * DHIST, DHIST)])


def _sc_deg(dstr, zeros16, ones16):
    mesh = plsc.VectorSubcoreMesh(core_axis_name="c", subcore_axis_name="s")
    fn = pl.kernel(
        _deg_body,
        mesh=mesh,
        compiler_params=_SC_PARAMS,
        out_type=jax.ShapeDtypeStruct((NZ, 2 * DHIST), jnp.float32),
        scratch_types=[
            pltpu.VMEM_SHARED((NZ, DHIST), jnp.float32),
            pltpu.VMEM((HGPT, CH), jnp.int32),
            pltpu.VMEM((CH, DHIST), jnp.float32),
            pltpu.SemaphoreType.DMA,
        ],
    )
    return fn(dstr, zeros16, ones16)


def _agg_body(y_hbm, srcr_hbm, dstr_hbm, zeros_hbm, z_out, y_sh, z_sh,
              idx_s, idx_d, rows0, rows1, rows2, rows3, sem0, sem1, sem2,
              sem3, sem4, sem5, sem6, sem7):
    c = lax.axis_index("c")
    s = lax.axis_index("s")
    r0 = s * RPT
    c0 = c * DH
    pltpu.sync_copy(zeros_hbm, z_sh.at[pl.ds(r0, RPT)])
    pltpu.sync_copy(y_hbm.at[pl.ds(s * NRS, NRS), pl.ds(c0, DH)],
                    y_sh.at[pl.ds(s * NRS, NRS)])

    @pl.when(s == NS - 1)
    def _():
        pltpu.sync_copy(
            y_hbm.at[pl.ds(NS * NRS, N - NS * NRS), pl.ds(c0, DH)],
            y_sh.at[pl.ds(NS * NRS, N - NS * NRS)])

    plsc.subcore_barrier()

    rows = [rows0, rows1, rows2, rows3]
    gsems = [sem0, sem1, sem2, sem3]
    ssems = [sem4, sem5, sem6, sem7]

    for h in range(NH):
        pltpu.sync_copy(srcr_hbm.at[s, h], idx_s)
        pltpu.sync_copy(dstr_hbm.at[s, h], idx_d)
        for k in range(3):
            pltpu.async_copy(y_sh.at[idx_s.at[k]], rows[k], gsems[k])

        def quad(i, carry):
            for k in range(4):
                g = 4 * i + k
                kn = (k + 3) % 4
                pltpu.make_async_copy(y_sh.at[idx_s.at[g]], rows[k],
                                      gsems[k]).wait()
                pltpu.async_copy(rows[k], z_sh.at[idx_d.at[g]], ssems[k],
                                 add=True)

                @pl.when((g >= 1) & (g + 3 < HGPT))
                def _():
                    pltpu.make_async_copy(rows[kn], z_sh.at[idx_d.at[g]],
                                          ssems[kn]).wait()

                @pl.when(g + 3 < HGPT)
                def _():
                    pltpu.async_copy(y_sh.at[idx_s.at[g + 3]], rows[kn],
                                     gsems[kn])
            return carry

        lax.fori_loop(0, HGPT // 4, quad, 0)
        for k in range(4):
            pltpu.make_async_copy(rows[k], z_sh.at[idx_d.at[0]],
                                  ssems[k]).wait()

    plsc.subcore_barrier()
    pltpu.sync_copy(z_sh.at[pl.ds(r0, RPT)],
                    z_out.at[pl.ds(r0, RPT), pl.ds(c0, DH)])


def _sc_agg(y, srcr, dstr, zeros64):
    mesh = plsc.VectorSubcoreMesh(core_axis_name="c", subcore_axis_name="s")
    fn = pl.kernel(
        _agg_body,
        mesh=mesh,
        compiler_params=_SC_PARAMS,
        out_type=jax.ShapeDtypeStruct((NZ, D), jnp.float32),
        scratch_types=[
            pltpu.VMEM_SHARED((N, DH), jnp.float32),
            pltpu.VMEM_SHARED((NZ, DH), jnp.float32),
            pltpu.VMEM((HGPT, CH), jnp.int32),
            pltpu.VMEM((HGPT, CH), jnp.int32),
            pltpu.VMEM((CH, DH), jnp.float32),
            pltpu.VMEM((CH, DH), jnp.float32),
            pltpu.VMEM((CH, DH), jnp.float32),
            pltpu.VMEM((CH, DH), jnp.float32),
            pltpu.SemaphoreType.DMA,
            pltpu.SemaphoreType.DMA,
            pltpu.SemaphoreType.DMA,
            pltpu.SemaphoreType.DMA,
            pltpu.SemaphoreType.DMA,
            pltpu.SemaphoreType.DMA,
            pltpu.SemaphoreType.DMA,
            pltpu.SemaphoreType.DMA,
        ],
    )
    return fn(y, srcr, dstr, zeros64)


def _prep_body(hist_ref, x_ref, w_ref, dinv_ref, y_ref):
    deg = hist_ref[:, 0:1] + hist_ref[:, DHIST:DHIST + 1] + 1.0
    dv = lax.rsqrt(deg)
    dinv_ref[...] = jnp.broadcast_to(dv, (BR, 16))
    y_ref[...] = jnp.dot(x_ref[...], w_ref[...],
                         preferred_element_type=jnp.float32) * dv


def _tc_prep(hist, x, w):
    return pl.pallas_call(
        _prep_body,
        grid=(N // BR,),
        in_specs=[
            pl.BlockSpec((BR, 2 * DHIST), lambda i: (i, 0)),
            pl.BlockSpec((BR, D), lambda i: (i, 0)),
            pl.BlockSpec((D, D), lambda i: (0, 0)),
        ],
        out_specs=[
            pl.BlockSpec((BR, 16), lambda i: (i, 0)),
            pl.BlockSpec((BR, D), lambda i: (i, 0)),
        ],
        out_shape=[
            jax.ShapeDtypeStruct((N, 16), jnp.float32),
            jax.ShapeDtypeStruct((N, D), jnp.float32),
        ],
    )(hist, x, w)


def _stage_body(z_ref, y_ref, dinv_ref, w_ref, b_ref, g_ref, be_ref,
                yo_ref):
    dv = dinv_ref[:, 0:1]
    t = (z_ref[...] + y_ref[...]) * dv + b_ref[...]
    h = jnp.maximum(t * (BN_S * g_ref[...]) + be_ref[...], 0.0)
    yo_ref[...] = jnp.dot(h, w_ref[...],
                          preferred_element_type=jnp.float32) * dv


def _tc_stage(z, y, dinv16, w_next, b, g, be):
    return pl.pallas_call(
        _stage_body,
        grid=(N // BR,),
        in_specs=[
            pl.BlockSpec((BR, D), lambda i: (i, 0)),
            pl.BlockSpec((BR, D), lambda i: (i, 0)),
            pl.BlockSpec((BR, 16), lambda i: (i, 0)),
            pl.BlockSpec((D, D), lambda i: (0, 0)),
            pl.BlockSpec((1, D), lambda i: (0, 0)),
            pl.BlockSpec((1, D), lambda i: (0, 0)),
            pl.BlockSpec((1, D), lambda i: (0, 0)),
        ],
        out_specs=pl.BlockSpec((BR, D), lambda i: (i, 0)),
        out_shape=jax.ShapeDtypeStruct((N, D), jnp.float32),
    )(z, y, dinv16, w_next, b.reshape(1, D), g.reshape(1, D),
      be.reshape(1, D))


def _final_body(z_ref, y_ref, dinv_ref, b_ref, o_ref):
    dv = dinv_ref[:, 0:1]
    o = (z_ref[...] + y_ref[...]) * dv + b_ref[...]
    m = jnp.max(o, axis=1, keepdims=True)
    lse = jnp.log(jnp.sum(jnp.exp(o - m), axis=1, keepdims=True)) + m
    o_ref[...] = o - lse


def _tc_final(z, y, dinv16, b):
    return pl.pallas_call(
        _final_body,
        grid=(N // BR,),
        in_specs=[
            pl.BlockSpec((BR, D), lambda i: (i, 0)),
            pl.BlockSpec((BR, D), lambda i: (i, 0)),
            pl.BlockSpec((BR, 16), lambda i: (i, 0)),
            pl.BlockSpec((1, D), lambda i: (0, 0)),
        ],
        out_specs=pl.BlockSpec((BR, D), lambda i: (i, 0)),
        out_shape=jax.ShapeDtypeStruct((N, D), jnp.float32),
    )(z, y, dinv16, b.reshape(1, D))


def kernel(x, edge_index, relations, W1, b1, g1, be1, W2, b2, g2, be2,
           W3, b3):
    del relations
    src = edge_index[0].reshape(NS, NH, HGPT, CH)
    dst = edge_index[1].reshape(NS, NH, HGPT, CH)
    zeros64 = jnp.zeros((RPT, DH), jnp.float32)
    zeros16 = jnp.zeros((RPT, DHIST), jnp.float32)
    ones16 = jnp.ones((CH, DHIST), jnp.float32)

    hist = _sc_deg(dst, zeros16, ones16)
    dinv16, y1 = _tc_prep(hist, x, W1)
    z1 = _sc_agg(y1, src, dst, zeros64)
    y2 = _tc_stage(z1, y1, dinv16, W2, b1, g1, be1)
    z2 = _sc_agg(y2, src, dst, zeros64)
    y3 = _tc_stage(z2, y2, dinv16, W3, b2, g2, be2)
    z3 = _sc_agg(y3, src, dst, zeros64)
    return _tc_final(z3, y3, dinv16, b3)
